# baseline (speedup 1.0000x reference)
import functools

import jax
import jax.numpy as jnp
from jax.experimental import pallas as pl
from jax.experimental.pallas import tpu as pltpu

B, S, I = 16, 32, 1024
H = 1024
M = 1024
D = 256
O = 1024
TOPK = 3


def _xw_body(x_ref, w_ref, b_ref, o_ref):
    o_ref[...] = (
        jax.lax.dot_general(
            x_ref[...], w_ref[...], (((1,), (1,)), ((), ())),
            preferred_element_type=jnp.float32,
        )
        + b_ref[...]
    )


def _compute_xw(x_sb, Wih, bsum):
    NBLK = 8
    blk = (4 * H) // NBLK
    return pl.pallas_call(
        _xw_body,
        grid=(NBLK,),
        in_specs=[
            pl.BlockSpec((S * B, I), lambda n: (0, 0)),
            pl.BlockSpec((blk, I), lambda n: (n, 0)),
            pl.BlockSpec((1, blk), lambda n: (0, n)),
        ],
        out_specs=pl.BlockSpec((S * B, blk), lambda n: (0, n)),
        out_shape=jax.ShapeDtypeStruct((S * B, 4 * H), jnp.float32),
    )(x_sb, Wih, bsum)


def _main_body(xw_ref, whh_ref, wq_ref, bq_ref, kmem_ref, vmem_ref, wa_ref,
               wc_ref, bc_ref, woh_ref, wod_ref, bo_ref, out_ref, co_ref):
    C = 256
    NCK = H // C

    c0_parts = []
    for ck in range(NCK):
        xi = xw_ref[0:B, 0 * H + ck * C:0 * H + (ck + 1) * C]
        xg = xw_ref[0:B, 2 * H + ck * C:2 * H + (ck + 1) * C]
        xo = xw_ref[0:B, 3 * H + ck * C:3 * H + (ck + 1) * C]
        c_ck = jax.nn.sigmoid(xi) * jnp.tanh(xg)
        co_ref[0:B, ck * C:(ck + 1) * C] = jax.nn.sigmoid(xo) * jnp.tanh(c_ck)
        c0_parts.append(c_ck)

    def step(t, c):
        hp = co_ref[pl.ds((t - 1) * B, B), :].astype(jnp.bfloat16)
        c_parts = []
        for ck in range(NCK):
            def gate(g):
                w = whh_ref[g * H + ck * C:g * H + (ck + 1) * C, :].astype(jnp.bfloat16)
                return xw_ref[pl.ds(t * B, B), g * H + ck * C:g * H + (ck + 1) * C] + \
                    jax.lax.dot_general(hp, w, (((1,), (1,)), ((), ())),
                                        preferred_element_type=jnp.float32)
            c_ck = (jax.nn.sigmoid(gate(1)) * c[:, ck * C:(ck + 1) * C]
                    + jax.nn.sigmoid(gate(0)) * jnp.tanh(gate(2)))
            co_ref[pl.ds(t * B, B), ck * C:(ck + 1) * C] = \
                jax.nn.sigmoid(gate(3)) * jnp.tanh(c_ck)
            c_parts.append(c_ck)
        return jnp.concatenate(c_parts, axis=1)

    jax.lax.fori_loop(1, S, step, jnp.concatenate(c0_parts, axis=1))

    co = co_ref[...]

    q = jax.lax.dot_general(co, wq_ref[...], (((1,), (1,)), ((), ())),
                            preferred_element_type=jnp.float32) + bq_ref[...]
    qn = q / jnp.maximum(jnp.sqrt(jnp.sum(q * q, axis=1, keepdims=True)), 1e-12)
    km = kmem_ref[...]
    kn = km / jnp.maximum(jnp.sqrt(jnp.sum(km * km, axis=1, keepdims=True)), 1e-12)
    sims = jax.lax.dot_general(qn, kn, (((1,), (1,)), ((), ())),
                               preferred_element_type=jnp.float32)

    vmem = vmem_ref[...]
    vl = jax.lax.dot_general(vmem, wa_ref[...], (((1,), (1,)), ((), ())),
                             preferred_element_type=jnp.float32)

    lane = jax.lax.broadcasted_iota(jnp.int32, (S * B, M), 1)
    retr = []
    logits = []
    for _ in range(TOPK):
        mx = jnp.max(sims, axis=1, keepdims=True)
        cand = jnp.where(sims >= mx, lane, M)
        sel = jnp.min(cand, axis=1, keepdims=True)
        onehot = (lane == sel).astype(jnp.float32)
        retr.append(jnp.dot(onehot, vmem, preferred_element_type=jnp.float32))
        logits.append(jnp.dot(onehot, vl, preferred_element_type=jnp.float32))
        sims = jnp.where(lane == sel, -jnp.inf, sims)

    lmax = jnp.maximum(jnp.maximum(logits[0], logits[1]), logits[2])
    e0 = jnp.exp(logits[0] - lmax)
    e1 = jnp.exp(logits[1] - lmax)
    e2 = jnp.exp(logits[2] - lmax)
    es = e0 + e1 + e2
    mem = (e0 * retr[0] + e1 * retr[1] + e2 * retr[2]) / es

    memc = jax.lax.dot_general(mem, wc_ref[...], (((1,), (1,)), ((), ())),
                               preferred_element_type=jnp.float32) + bc_ref[...]
    out_ref[...] = (
        jax.lax.dot_general(co, woh_ref[...], (((1,), (1,)), ((), ())),
                            preferred_element_type=jnp.float32)
        + jax.lax.dot_general(memc, wod_ref[...], (((1,), (1,)), ((), ())),
                              preferred_element_type=jnp.float32)
        + bo_ref[...]
    )


def kernel(x, Wih, Whh, bih, bhh, Wq, bq, Wa, ba, Wc, bc, Wo, bo, Kmem, Vmem):
    x_sb = jnp.transpose(x, (1, 0, 2)).reshape(S * B, I)
    bsum = (bih + bhh).reshape(1, 4 * H)
    xw = _compute_xw(x_sb, Wih, bsum)

    out_flat = pl.pallas_call(
        _main_body,
        out_shape=jax.ShapeDtypeStruct((S * B, O), jnp.float32),
        scratch_shapes=[pltpu.VMEM((S * B, H), jnp.float32)],
    )(xw, Whh, Wq, bq.reshape(1, D), Kmem, Vmem, Wa, Wc, bc.reshape(1, D),
      Wo[:, :H], Wo[:, H:], bo.reshape(1, O))

    return jnp.transpose(out_flat.reshape(S, B, O), (1, 0, 2))


# speedup vs baseline: 1.0196x; 1.0196x over previous
import functools

import jax
import jax.numpy as jnp
from jax.experimental import pallas as pl
from jax.experimental.pallas import tpu as pltpu

B, S, I = 16, 32, 1024
H = 1024
M = 1024
D = 256
O = 1024
TOPK = 3


def _xw_body(x_ref, w_ref, b_ref, o_ref):
    o_ref[...] = (
        jax.lax.dot_general(
            x_ref[...], w_ref[...], (((1,), (1,)), ((), ())),
            preferred_element_type=jnp.float32,
        )
        + b_ref[...]
    )


def _compute_xw(x_sb, Wih, bsum):
    NBLK = 8
    blk = (4 * H) // NBLK
    return pl.pallas_call(
        _xw_body,
        grid=(NBLK,),
        in_specs=[
            pl.BlockSpec((S * B, I), lambda n: (0, 0)),
            pl.BlockSpec((blk, I), lambda n: (n, 0)),
            pl.BlockSpec((1, blk), lambda n: (0, n)),
        ],
        out_specs=pl.BlockSpec((S * B, blk), lambda n: (0, n)),
        out_shape=jax.ShapeDtypeStruct((S * B, 4 * H), jnp.float32),
    )(x_sb, Wih, bsum)


def _main_body(xw_ref, whh_ref, wq_ref, bq_ref, kmem_ref, vmem_ref, wa_ref,
               wc_ref, bc_ref, woh_ref, wod_ref, bo_ref, out_ref, co_ref):
    C = 256
    NCK = H // C

    c0_parts = []
    for ck in range(NCK):
        xi = xw_ref[0:B, 0 * H + ck * C:0 * H + (ck + 1) * C]
        xg = xw_ref[0:B, 2 * H + ck * C:2 * H + (ck + 1) * C]
        xo = xw_ref[0:B, 3 * H + ck * C:3 * H + (ck + 1) * C]
        c_ck = jax.nn.sigmoid(xi) * jnp.tanh(xg)
        co_ref[0:B, ck * C:(ck + 1) * C] = jax.nn.sigmoid(xo) * jnp.tanh(c_ck)
        c0_parts.append(c_ck)

    c_parts = c0_parts
    for t in range(1, S):
        hp = co_ref[(t - 1) * B:t * B, :]
        new_parts = []
        for ck in range(NCK):
            def gate(g):
                w = whh_ref[g * H + ck * C:g * H + (ck + 1) * C, :]
                return xw_ref[t * B:(t + 1) * B, g * H + ck * C:g * H + (ck + 1) * C] + \
                    jax.lax.dot_general(hp, w, (((1,), (1,)), ((), ())),
                                        preferred_element_type=jnp.float32)
            c_ck = (jax.nn.sigmoid(gate(1)) * c_parts[ck]
                    + jax.nn.sigmoid(gate(0)) * jnp.tanh(gate(2)))
            co_ref[t * B:(t + 1) * B, ck * C:(ck + 1) * C] = \
                jax.nn.sigmoid(gate(3)) * jnp.tanh(c_ck)
            new_parts.append(c_ck)
        c_parts = new_parts

    co = co_ref[...]

    q = jax.lax.dot_general(co, wq_ref[...], (((1,), (1,)), ((), ())),
                            preferred_element_type=jnp.float32) + bq_ref[...]
    qn = q / jnp.maximum(jnp.sqrt(jnp.sum(q * q, axis=1, keepdims=True)), 1e-12)
    km = kmem_ref[...]
    kn = km / jnp.maximum(jnp.sqrt(jnp.sum(km * km, axis=1, keepdims=True)), 1e-12)
    sims = jax.lax.dot_general(qn, kn, (((1,), (1,)), ((), ())),
                               preferred_element_type=jnp.float32)

    vmem = vmem_ref[...]
    vl = jax.lax.dot_general(vmem, wa_ref[...], (((1,), (1,)), ((), ())),
                             preferred_element_type=jnp.float32)

    lane = jax.lax.broadcasted_iota(jnp.int32, (S * B, M), 1)
    retr = []
    logits = []
    for _ in range(TOPK):
        mx = jnp.max(sims, axis=1, keepdims=True)
        cand = jnp.where(sims >= mx, lane, M)
        sel = jnp.min(cand, axis=1, keepdims=True)
        onehot = (lane == sel).astype(jnp.float32)
        retr.append(jnp.dot(onehot, vmem, preferred_element_type=jnp.float32))
        logits.append(jnp.dot(onehot, vl, preferred_element_type=jnp.float32))
        sims = jnp.where(lane == sel, -jnp.inf, sims)

    lmax = jnp.maximum(jnp.maximum(logits[0], logits[1]), logits[2])
    e0 = jnp.exp(logits[0] - lmax)
    e1 = jnp.exp(logits[1] - lmax)
    e2 = jnp.exp(logits[2] - lmax)
    es = e0 + e1 + e2
    mem = (e0 * retr[0] + e1 * retr[1] + e2 * retr[2]) / es

    memc = jax.lax.dot_general(mem, wc_ref[...], (((1,), (1,)), ((), ())),
                               preferred_element_type=jnp.float32) + bc_ref[...]
    out_ref[...] = (
        jax.lax.dot_general(co, woh_ref[...], (((1,), (1,)), ((), ())),
                            preferred_element_type=jnp.float32)
        + jax.lax.dot_general(memc, wod_ref[...], (((1,), (1,)), ((), ())),
                              preferred_element_type=jnp.float32)
        + bo_ref[...]
    )


def kernel(x, Wih, Whh, bih, bhh, Wq, bq, Wa, ba, Wc, bc, Wo, bo, Kmem, Vmem):
    x_sb = jnp.transpose(x, (1, 0, 2)).reshape(S * B, I)
    bsum = (bih + bhh).reshape(1, 4 * H)
    xw = _compute_xw(x_sb, Wih, bsum)

    out_flat = pl.pallas_call(
        _main_body,
        out_shape=jax.ShapeDtypeStruct((S * B, O), jnp.float32),
        scratch_shapes=[pltpu.VMEM((S * B, H), jnp.float32)],
    )(xw, Whh, Wq, bq.reshape(1, D), Kmem, Vmem, Wa, Wc, bc.reshape(1, D),
      Wo[:, :H], Wo[:, H:], bo.reshape(1, O))

    return jnp.transpose(out_flat.reshape(S, B, O), (1, 0, 2))
